# Initial kernel scaffold; baseline (speedup 1.0000x reference)
import jax
import jax.numpy as jnp
from jax.experimental import pallas as pl


def kernel(speaker_ids, init_speaker_ids, audio_encodings):
    raise NotImplementedError("write your pallas kernel here")


# speedup vs baseline: 2.4002x; 2.4002x over previous
import functools

import jax
import jax.numpy as jnp
from jax import lax
from jax.experimental import pallas as pl
from jax.experimental.pallas import tpu as pltpu
from jax.experimental.pallas import tpu_sc as plsc

NUM_SPEAKERS = 1000
NUM_UTTER = 16384
EMBED_DIM = 128
BATCH = 16384

NC = 2
NS = 16
COLS = EMBED_DIM // NC
ROWS_PER_TILE = NUM_UTTER // NS
CHUNK = 128
NCHUNK = ROWS_PER_TILE // CHUNK
TABLE_ROWS = 1024


def _sc_body(audio_hbm, init_idx_hbm, q_idx_hbm, out_hbm,
             rowbuf, init_idx_v, q_idx_v, table):
    c = lax.axis_index("c")
    s = lax.axis_index("s")
    c0 = c * COLS
    base = s * ROWS_PER_TILE
    idx_row0 = s * NCHUNK

    zrows = TABLE_ROWS // NS
    zvec = jnp.zeros((16,), jnp.float32)

    def _zero_row(r, _):
        for cc in range(COLS // 16):
            rowbuf[r, pl.ds(cc * 16, 16)] = zvec
        return _

    lax.fori_loop(0, zrows, _zero_row, 0)
    pltpu.sync_copy(rowbuf.at[pl.ds(0, zrows)], table.at[pl.ds(s * zrows, zrows)])

    pltpu.sync_copy(init_idx_hbm.at[pl.ds(idx_row0, NCHUNK)], init_idx_v)
    pltpu.sync_copy(q_idx_hbm.at[pl.ds(idx_row0, NCHUNK)], q_idx_v)
    pltpu.sync_copy(audio_hbm.at[pl.ds(base, ROWS_PER_TILE), pl.ds(c0, COLS)],
                    rowbuf)

    plsc.subcore_barrier()

    for j in range(NCHUNK):
        pltpu.sync_copy(rowbuf.at[pl.ds(j * CHUNK, CHUNK)],
                        table.at[init_idx_v.at[j]], add=True)

    plsc.subcore_barrier()

    for j in range(NCHUNK):
        pltpu.sync_copy(table.at[q_idx_v.at[j]],
                        rowbuf.at[pl.ds(j * CHUNK, CHUNK)])
    pltpu.sync_copy(rowbuf,
                    out_hbm.at[pl.ds(base, ROWS_PER_TILE), pl.ds(c0, COLS)])


@functools.partial(jax.jit, static_argnums=())
def _sc_call(audio, init_idx2d, q_idx2d):
    mesh = plsc.VectorSubcoreMesh(core_axis_name="c", subcore_axis_name="s")
    f = functools.partial(
        pl.kernel,
        mesh=mesh,
        out_type=jax.ShapeDtypeStruct((BATCH, EMBED_DIM), jnp.float32),
        scratch_types=[
            pltpu.VMEM((ROWS_PER_TILE, COLS), jnp.float32),
            pltpu.VMEM((NCHUNK, CHUNK), jnp.int32),
            pltpu.VMEM((NCHUNK, CHUNK), jnp.int32),
            pltpu.VMEM_SHARED((TABLE_ROWS, COLS), jnp.float32),
        ],
        compiler_params=pltpu.CompilerParams(use_tc_tiling_on_sc=False),
    )(_sc_body)
    return f(audio, init_idx2d, q_idx2d)


def kernel(speaker_ids, init_speaker_ids, audio_encodings):
    q2d = speaker_ids.astype(jnp.int32).reshape(NS * NCHUNK, CHUNK)
    i2d = init_speaker_ids.astype(jnp.int32).reshape(NS * NCHUNK, CHUNK)
    return _sc_call(audio_encodings, i2d, q2d)


# speedup vs baseline: 2.7640x; 1.1515x over previous
import functools

import jax
import jax.numpy as jnp
from jax import lax
from jax.experimental import pallas as pl
from jax.experimental.pallas import tpu as pltpu
from jax.experimental.pallas import tpu_sc as plsc

NUM_SPEAKERS = 1000
NUM_UTTER = 16384
EMBED_DIM = 128
BATCH = 16384

NC = 2
NS = 16
COLS = EMBED_DIM // NC
ROWS_PER_TILE = NUM_UTTER // NS
CHUNK = 128
NCHUNK = ROWS_PER_TILE // CHUNK
TABLE_ROWS = 1024


def _sc_body(audio_hbm, init_idx_hbm, q_idx_hbm, out_hbm,
             rowbuf, zbuf, init_idx_v, q_idx_v, table,
             load_sems, gat_sems, scat_sem, wr_sem, idx_sem_i, idx_sem_q):
    c = lax.axis_index("c")
    s = lax.axis_index("s")
    c0 = c * COLS
    base = s * ROWS_PER_TILE
    idx_row0 = s * NCHUNK

    idx_cp_i = pltpu.async_copy(
        init_idx_hbm.at[pl.ds(idx_row0, NCHUNK)], init_idx_v, idx_sem_i)
    idx_cp_q = pltpu.async_copy(
        q_idx_hbm.at[pl.ds(idx_row0, NCHUNK)], q_idx_v, idx_sem_q)
    loads = [
        pltpu.async_copy(
            audio_hbm.at[pl.ds(base + j * CHUNK, CHUNK), pl.ds(c0, COLS)],
            rowbuf.at[pl.ds(j * CHUNK, CHUNK)], load_sems.at[j])
        for j in range(NCHUNK)
    ]

    zrows = TABLE_ROWS // NS
    zvec = jnp.zeros((16,), jnp.float32)

    def _zero_row(r, _):
        for cc in range(COLS // 16):
            zbuf[r, pl.ds(cc * 16, 16)] = zvec
        return _

    lax.fori_loop(0, zrows, _zero_row, 0)
    pltpu.sync_copy(zbuf, table.at[pl.ds(s * zrows, zrows)])

    idx_cp_i.wait()
    plsc.subcore_barrier()

    for j in range(NCHUNK):
        loads[j].wait()
        pltpu.sync_copy(rowbuf.at[pl.ds(j * CHUNK, CHUNK)],
                        table.at[init_idx_v.at[j]], add=True)
    idx_cp_q.wait()

    plsc.subcore_barrier()

    gathers = [
        pltpu.async_copy(table.at[q_idx_v.at[j]],
                         rowbuf.at[pl.ds(j * CHUNK, CHUNK)], gat_sems.at[j])
        for j in range(NCHUNK)
    ]
    writes = []
    for j in range(NCHUNK):
        gathers[j].wait()
        writes.append(pltpu.async_copy(
            rowbuf.at[pl.ds(j * CHUNK, CHUNK)],
            out_hbm.at[pl.ds(base + j * CHUNK, CHUNK), pl.ds(c0, COLS)],
            wr_sem))
    for cp in writes:
        cp.wait()


@functools.partial(jax.jit, static_argnums=())
def _sc_call(audio, init_idx2d, q_idx2d):
    mesh = plsc.VectorSubcoreMesh(core_axis_name="c", subcore_axis_name="s")
    f = functools.partial(
        pl.kernel,
        mesh=mesh,
        out_type=jax.ShapeDtypeStruct((BATCH, EMBED_DIM), jnp.float32),
        scratch_types=[
            pltpu.VMEM((ROWS_PER_TILE, COLS), jnp.float32),
            pltpu.VMEM((TABLE_ROWS // NS, COLS), jnp.float32),
            pltpu.VMEM((NCHUNK, CHUNK), jnp.int32),
            pltpu.VMEM((NCHUNK, CHUNK), jnp.int32),
            pltpu.VMEM_SHARED((TABLE_ROWS, COLS), jnp.float32),
            pltpu.SemaphoreType.DMA((NCHUNK,)),
            pltpu.SemaphoreType.DMA((NCHUNK,)),
            pltpu.SemaphoreType.DMA,
            pltpu.SemaphoreType.DMA,
            pltpu.SemaphoreType.DMA,
            pltpu.SemaphoreType.DMA,
        ],
        compiler_params=pltpu.CompilerParams(use_tc_tiling_on_sc=False),
    )(_sc_body)
    return f(audio, init_idx2d, q_idx2d)


def kernel(speaker_ids, init_speaker_ids, audio_encodings):
    q2d = speaker_ids.astype(jnp.int32).reshape(NS * NCHUNK, CHUNK)
    i2d = init_speaker_ids.astype(jnp.int32).reshape(NS * NCHUNK, CHUNK)
    return _sc_call(audio_encodings, i2d, q2d)
